# baseline (speedup 1.0000x reference)
import functools

import jax
import jax.numpy as jnp
from jax import lax
from jax.experimental import pallas as pl
from jax.experimental.pallas import tpu as pltpu
from jax.experimental.pallas import tpu_sc as plsc

_NC = 2
_NS = 16
_NW = _NC * _NS
_L = 16
_B = 1024
_D = 64
_BPW = _B // _NW


@functools.partial(
    pl.kernel,
    mesh=plsc.VectorSubcoreMesh(core_axis_name="c", subcore_axis_name="s"),
    out_type=jax.ShapeDtypeStruct((_NW, _L), jnp.float32),
    compiler_params=pltpu.CompilerParams(use_tc_tiling_on_sc=False),
    scratch_types=[
        pltpu.VMEM((_BPW,), jnp.int32),
        pltpu.VMEM((_BPW, _D), jnp.float32),
        pltpu.VMEM((_BPW, _D), jnp.float32),
        pltpu.VMEM((_L,), jnp.float32),
        pltpu.SemaphoreType.DMA,
    ],
)
def _center_mse_partials(features_hbm, labels_hbm, centers_hbm, out_hbm,
                         idx_v, feat_v, rows_v, acc_v, sem):
    wid = lax.axis_index("s") * _NC + lax.axis_index("c")
    base = wid * _BPW
    pltpu.sync_copy(labels_hbm.at[pl.ds(base, _BPW)], idx_v)
    gather = pltpu.async_copy(centers_hbm.at[idx_v], rows_v, sem)
    pltpu.sync_copy(features_hbm.at[pl.ds(base, _BPW)], feat_v)
    gather.wait()
    acc = jnp.zeros((_L,), jnp.float32)
    for i in range(_BPW):
        for j in range(_D // _L):
            d = feat_v[i, pl.ds(j * _L, _L)] - rows_v[i, pl.ds(j * _L, _L)]
            acc = acc + d * d
    acc_v[...] = acc
    pltpu.sync_copy(acc_v, out_hbm.at[wid])


def kernel(features, labels, centers):
    partials = _center_mse_partials(
        features, labels.astype(jnp.int32), centers)
    return jnp.sum(partials) / jnp.float32(_B * _D)


# speedup vs baseline: 1.6624x; 1.6624x over previous
import functools

import jax
import jax.numpy as jnp
from jax import lax
from jax.experimental import pallas as pl
from jax.experimental.pallas import tpu as pltpu
from jax.experimental.pallas import tpu_sc as plsc

_NC = 2
_NS = 16
_NW = _NC * _NS
_L = 16
_B = 1024
_D = 64
_R = 8
_BPW = _B // _NW


@functools.partial(
    pl.kernel,
    mesh=plsc.VectorSubcoreMesh(core_axis_name="c", subcore_axis_name="s"),
    out_type=jax.ShapeDtypeStruct((_NW, _L), jnp.float32),
    compiler_params=pltpu.CompilerParams(needs_layout_passes=False),
    scratch_types=[
        pltpu.VMEM((_BPW,), jnp.int32),
        pltpu.VMEM((_BPW,), jnp.int32),
        pltpu.VMEM((_BPW, _D), jnp.float32),
        pltpu.VMEM((_BPW, _R, _D), jnp.float32),
        pltpu.VMEM((_L,), jnp.float32),
        pltpu.SemaphoreType.DMA,
    ],
)
def _center_mse_partials(features_hbm, labels_hbm, centers_hbm, out_hbm,
                         idx_v, tid_v, feat_v, rows_v, acc_v, sem):
    wid = lax.axis_index("s") * _NC + lax.axis_index("c")
    base = wid * _BPW
    pltpu.sync_copy(labels_hbm.at[pl.ds(base, _BPW)], idx_v)
    copies = []
    for c in range(_BPW // _L):
        lbl = idx_v[pl.ds(c * _L, _L)]
        tid = lax.shift_right_logical(lbl, 3)
        for k in range(_L):
            copies.append(pltpu.async_copy(
                centers_hbm.at[tid[k]], rows_v.at[c * _L + k], sem))
    pltpu.sync_copy(features_hbm.at[pl.ds(base, _BPW)], feat_v)
    for cp in copies:
        cp.wait()
    row_iota = lax.iota(jnp.int32, _L)
    acc = jnp.zeros((_L,), jnp.float32)
    for c in range(_BPW // _L):
        lbl = idx_v[pl.ds(c * _L, _L)]
        sub = lax.bitwise_and(lbl, jnp.full((_L,), _R - 1, jnp.int32))
        d0 = row_iota + c * _L
        for j in range(_D):
            col = jnp.full((_L,), j, jnp.int32)
            cv = plsc.load_gather(rows_v, [d0, sub, col])
            fv = plsc.load_gather(feat_v, [d0, col])
            d = fv - cv
            acc = acc + d * d
    acc_v[...] = acc
    pltpu.sync_copy(acc_v, out_hbm.at[wid])


def kernel(features, labels, centers):
    centers3 = centers.reshape(centers.shape[0] // _R, _R, _D)
    partials = _center_mse_partials(
        features, labels.astype(jnp.int32), centers3)
    return jnp.sum(partials) / jnp.float32(_B * _D)
